# baseline (speedup 1.0000x reference)
import functools

import jax
import jax.numpy as jnp
from jax.experimental import pallas as pl
from jax.experimental.pallas import tpu as pltpu

_VMEM_LIMIT = 64 * 1024 * 1024

_DN_T = (((1,), (1,)), ((), ()))


def _fused_mha_kernel(x_ref, wq_ref, wk_ref, wv_ref, wo_ref, o_ref,
                      wqkv_s, wo_s,
                      *, num_heads, head_dim, scaling):
    f32 = jnp.float32
    cdt = jnp.bfloat16
    e = x_ref.shape[1]

    @pl.when(pl.program_id(0) == 0)
    def _():
        wqkv_s[0:e, :] = (wq_ref[...] * scaling).astype(cdt)
        wqkv_s[e:2 * e, :] = wk_ref[...].astype(cdt)
        wqkv_s[2 * e:3 * e, :] = wv_ref[...].astype(cdt)
        wo_s[...] = wo_ref[...].astype(cdt)

    x = x_ref[...].astype(cdt)

    qkv = jax.lax.dot_general(x, wqkv_s[...], _DN_T,
                              preferred_element_type=f32).astype(cdt)

    outs = []
    for h in range(num_heads):
        sl = slice(h * head_dim, (h + 1) * head_dim)
        qh = qkv[:, h * head_dim:(h + 1) * head_dim]
        kh = qkv[:, e + h * head_dim:e + (h + 1) * head_dim]
        vh = qkv[:, 2 * e + h * head_dim:2 * e + (h + 1) * head_dim]
        s = jax.lax.dot_general(qh, kh, _DN_T,
                                preferred_element_type=f32)
        m = jnp.max(s, axis=-1, keepdims=True)
        p = jnp.exp(s - m)
        l = jnp.sum(p, axis=-1, keepdims=True)
        acc = jnp.dot(p.astype(cdt), vh, preferred_element_type=f32)
        outs.append((acc * pl.reciprocal(l, approx=False)).astype(cdt))

    attn = jnp.concatenate(outs, axis=-1)

    out = jax.lax.dot_general(attn, wo_s[...], _DN_T,
                              preferred_element_type=f32)
    o_ref[...] = out.astype(cdt).astype(o_ref.dtype)


def kernel(hidden_states, wq, wk, wv, wo):
    B, T, E = hidden_states.shape
    num_heads = 16
    head_dim = E // num_heads
    scaling = head_dim ** (-0.5)
    orig_dtype = hidden_states.dtype
    cdt = jnp.bfloat16

    cost = pl.CostEstimate(
        flops=2 * B * T * E * E * 4 + 4 * B * num_heads * T * T * head_dim,
        transcendentals=B * num_heads * T * T,
        bytes_accessed=B * T * E * 8 + 4 * E * E * 4,
    )

    fused = functools.partial(
        _fused_mha_kernel, num_heads=num_heads, head_dim=head_dim,
        scaling=scaling)

    out = pl.pallas_call(
        fused,
        out_shape=jax.ShapeDtypeStruct((B, T, E), orig_dtype),
        grid_spec=pltpu.PrefetchScalarGridSpec(
            num_scalar_prefetch=0,
            grid=(B,),
            in_specs=[
                pl.BlockSpec((None, T, E), lambda b: (b, 0, 0)),
                pl.BlockSpec((E, E), lambda b: (0, 0)),
                pl.BlockSpec((E, E), lambda b: (0, 0)),
                pl.BlockSpec((E, E), lambda b: (0, 0)),
                pl.BlockSpec((E, E), lambda b: (0, 0)),
            ],
            out_specs=pl.BlockSpec((None, T, E), lambda b: (b, 0, 0)),
            scratch_shapes=[
                pltpu.VMEM((3 * E, E), cdt),
                pltpu.VMEM((E, E), cdt),
            ],
        ),
        compiler_params=pltpu.CompilerParams(
            dimension_semantics=("arbitrary",),
            vmem_limit_bytes=_VMEM_LIMIT,
        ),
        cost_estimate=cost,
    )(hidden_states, wq, wk, wv, wo)
    return out


# speedup vs baseline: 1.0014x; 1.0014x over previous
import functools

import jax
import jax.numpy as jnp
from jax.experimental import pallas as pl
from jax.experimental.pallas import tpu as pltpu

_VMEM_LIMIT = 64 * 1024 * 1024

_DN_T = (((1,), (1,)), ((), ()))


def _fused_mha_kernel(x_ref, wq_ref, wk_ref, wv_ref, wo_ref, o_ref,
                      wqkv_s, wo_s,
                      *, num_heads, head_dim, scaling):
    f32 = jnp.float32
    cdt = jnp.bfloat16
    e = x_ref.shape[1]

    @pl.when(pl.program_id(0) == 0)
    def _():
        wqkv_s[0:e, :] = (wq_ref[...] * scaling).astype(cdt)
        wqkv_s[e:2 * e, :] = wk_ref[...].astype(cdt)
        wqkv_s[2 * e:3 * e, :] = wv_ref[...].astype(cdt)
        wo_s[...] = wo_ref[...].astype(cdt)

    x = x_ref[...].astype(cdt)

    qkv = jax.lax.dot_general(x, wqkv_s[...], _DN_T,
                              preferred_element_type=f32).astype(cdt)

    outs = []
    for h in range(num_heads):
        sl = slice(h * head_dim, (h + 1) * head_dim)
        qh = qkv[:, h * head_dim:(h + 1) * head_dim]
        kh = qkv[:, e + h * head_dim:e + (h + 1) * head_dim]
        vh = qkv[:, 2 * e + h * head_dim:2 * e + (h + 1) * head_dim]
        s = jax.lax.dot_general(qh, kh, _DN_T,
                                preferred_element_type=f32)
        m = jnp.max(s, axis=-1, keepdims=True)
        p = jnp.exp(s - m)
        l = jnp.sum(p, axis=-1, keepdims=True)
        acc = jnp.dot(p.astype(cdt), vh, preferred_element_type=f32)
        outs.append((acc * pl.reciprocal(l, approx=False)).astype(cdt))

    attn = jnp.concatenate(outs, axis=-1)

    out = jax.lax.dot_general(attn, wo_s[...], _DN_T,
                              preferred_element_type=f32)
    o_ref[...] = out.astype(o_ref.dtype)


def kernel(hidden_states, wq, wk, wv, wo):
    B, T, E = hidden_states.shape
    num_heads = 16
    head_dim = E // num_heads
    scaling = head_dim ** (-0.5)
    orig_dtype = hidden_states.dtype
    cdt = jnp.bfloat16

    cost = pl.CostEstimate(
        flops=2 * B * T * E * E * 4 + 4 * B * num_heads * T * T * head_dim,
        transcendentals=B * num_heads * T * T,
        bytes_accessed=B * T * E * 8 + 4 * E * E * 4,
    )

    fused = functools.partial(
        _fused_mha_kernel, num_heads=num_heads, head_dim=head_dim,
        scaling=scaling)

    out = pl.pallas_call(
        fused,
        out_shape=jax.ShapeDtypeStruct((B, T, E), orig_dtype),
        grid_spec=pltpu.PrefetchScalarGridSpec(
            num_scalar_prefetch=0,
            grid=(B,),
            in_specs=[
                pl.BlockSpec((None, T, E), lambda b: (b, 0, 0)),
                pl.BlockSpec((E, E), lambda b: (0, 0)),
                pl.BlockSpec((E, E), lambda b: (0, 0)),
                pl.BlockSpec((E, E), lambda b: (0, 0)),
                pl.BlockSpec((E, E), lambda b: (0, 0)),
            ],
            out_specs=pl.BlockSpec((None, T, E), lambda b: (b, 0, 0)),
            scratch_shapes=[
                pltpu.VMEM((3 * E, E), cdt),
                pltpu.VMEM((E, E), cdt),
            ],
        ),
        compiler_params=pltpu.CompilerParams(
            dimension_semantics=("arbitrary",),
            vmem_limit_bytes=_VMEM_LIMIT,
        ),
        cost_estimate=cost,
    )(hidden_states, wq, wk, wv, wo)
    return out
